# Initial kernel scaffold; baseline (speedup 1.0000x reference)
import jax
import jax.numpy as jnp
from jax.experimental import pallas as pl


def kernel(cls_preds, reg_preds, anchors, gt_boxes, gt_labels):
    raise NotImplementedError("write your pallas kernel here")


# speedup vs baseline: 2.4736x; 2.4736x over previous
import jax
import jax.numpy as jnp
from jax.experimental import pallas as pl
from jax.experimental.pallas import tpu as pltpu

N_ANCH = 20000
N_CLS = 80
N_BINS = 16
TOP_K = 9
STRIDE = 8.0
MPAD = 128
BN0 = 400
NBLK0 = N_ANCH // BN0
BNM = 1000
NBLKM = N_ANCH // BNM
BIGF = 3.0e38
BIGI = 2**30


def _dist_iou(ax0, ay0, ax1, ay1, gx0, gy0, gx1, gy1):
    acx = (ax0 + ax1) * 0.5
    acy = (ay0 + ay1) * 0.5
    gcx = (gx0 + gx1) * 0.5
    gcy = (gy0 + gy1) * 0.5
    dx = acx - gcx
    dy = acy - gcy
    dist = jnp.sqrt(dx * dx + dy * dy)
    area_a = (ax1 - ax0) * (ay1 - ay0)
    area_g = (gx1 - gx0) * (gy1 - gy0)
    ltx = jnp.maximum(ax0, gx0)
    lty = jnp.maximum(ay0, gy0)
    rbx = jnp.minimum(ax1, gx1)
    rby = jnp.minimum(ay1, gy1)
    iw = jnp.maximum(rbx - ltx, 0.0)
    ih = jnp.maximum(rby - lty, 0.0)
    inter = iw * ih
    union = area_a + area_g - inter
    iou = inter / jnp.maximum(union, 1e-9)
    return dist, iou, acx, acy


def _thresh_kernel(a_ref, gt_ref, out_ref, dcand, icand, ucand):
    b = pl.program_id(0)

    @pl.when(b == 0)
    def _init():
        dcand[...] = jnp.full(dcand.shape, BIGF, jnp.float32)
        icand[...] = jnp.full(icand.shape, BIGI, jnp.int32)
        ucand[...] = jnp.zeros(ucand.shape, jnp.float32)

    ax0 = a_ref[:, 0:1]
    ay0 = a_ref[:, 1:2]
    ax1 = a_ref[:, 2:3]
    ay1 = a_ref[:, 3:4]
    gx0 = gt_ref[0:1, :]
    gy0 = gt_ref[1:2, :]
    gx1 = gt_ref[2:3, :]
    gy1 = gt_ref[3:4, :]
    dist, iou, _, _ = _dist_iou(ax0, ay0, ax1, ay1, gx0, gy0, gx1, gy1)
    rows = jax.lax.broadcasted_iota(jnp.int32, (BN0, MPAD), 0) + b * BN0
    d = dist
    for r in range(TOP_K):
        mv = jnp.min(d, axis=0, keepdims=True)
        ism = d == mv
        midx = jnp.min(jnp.where(ism, rows, BIGI), axis=0, keepdims=True)
        sel = rows == midx
        uv = jnp.sum(jnp.where(sel, iou, 0.0), axis=0, keepdims=True)
        dcand[b, r : r + 1, :] = mv
        icand[b, r : r + 1, :] = midx
        ucand[b, r : r + 1, :] = uv
        d = jnp.where(sel, BIGF, d)

    @pl.when(b == NBLK0 - 1)
    def _merge():
        dd = dcand[...].reshape(NBLK0 * 16, MPAD)
        ii = icand[...].reshape(NBLK0 * 16, MPAD)
        uu = ucand[...].reshape(NBLK0 * 16, MPAD)
        picks = []
        d2 = dd
        for _ in range(TOP_K):
            mv = jnp.min(d2, axis=0, keepdims=True)
            tie = d2 == mv
            midx = jnp.min(jnp.where(tie, ii, BIGI), axis=0, keepdims=True)
            sel = tie & (ii == midx)
            uv = jnp.sum(jnp.where(sel, uu, 0.0), axis=0, keepdims=True)
            picks.append(uv)
            d2 = jnp.where(sel, BIGF, d2)
        s = picks[0]
        for x in picks[1:]:
            s = s + x
        mean = s / jnp.float32(TOP_K)
        var = (picks[0] - mean) ** 2
        for x in picks[1:]:
            var = var + (x - mean) ** 2
        std = jnp.sqrt(var / jnp.float32(TOP_K - 1))
        out_ref[...] = jnp.broadcast_to(mean + std, out_ref.shape)


def _thresh_call(anchors, gt_t):
    return pl.pallas_call(
        _thresh_kernel,
        grid=(NBLK0,),
        in_specs=[
            pl.BlockSpec((BN0, 4), lambda b: (b, 0)),
            pl.BlockSpec((8, MPAD), lambda b: (0, 0)),
        ],
        out_specs=pl.BlockSpec((8, MPAD), lambda b: (0, 0)),
        out_shape=jax.ShapeDtypeStruct((8, MPAD), jnp.float32),
        scratch_shapes=[
            pltpu.VMEM((NBLK0, 16, MPAD), jnp.float32),
            pltpu.VMEM((NBLK0, 16, MPAD), jnp.int32),
            pltpu.VMEM((NBLK0, 16, MPAD), jnp.float32),
        ],
        compiler_params=pltpu.CompilerParams(
            dimension_semantics=("arbitrary",)),
    )(anchors, gt_t)


def _main_kernel(a_ref, cls_ref, reg_ref, gt_ref, lab_ref, th_ref, out_ref):
    i = pl.program_id(0)
    ax0 = a_ref[:, 0:1]
    ay0 = a_ref[:, 1:2]
    ax1 = a_ref[:, 2:3]
    ay1 = a_ref[:, 3:4]
    gx0 = gt_ref[0:1, :]
    gy0 = gt_ref[1:2, :]
    gx1 = gt_ref[2:3, :]
    gy1 = gt_ref[3:4, :]
    _, iou, acx, acy = _dist_iou(ax0, ay0, ax1, ay1, gx0, gy0, gx1, gy1)
    thresh = th_ref[0:1, :]
    labrow = lab_ref[0:1, :]
    inside = ((acx >= gx0) & (acx <= gx1) & (acy >= gy0) & (acy <= gy1))
    pos = (iou >= thresh) & inside
    lanem = jax.lax.broadcasted_iota(jnp.int32, (BNM, MPAD), 1)
    matched = jnp.max(jnp.where(pos, lanem, -1), axis=1, keepdims=True)
    sel = lanem == matched
    posf = (matched >= 0).astype(jnp.float32)
    maxiou = jnp.sum(jnp.where(sel, iou, 0.0), axis=1, keepdims=True)
    label = jnp.sum(jnp.where(sel, labrow, 0), axis=1, keepdims=True)
    tbx0 = jnp.sum(jnp.where(sel, gx0, 0.0), axis=1, keepdims=True)
    tby0 = jnp.sum(jnp.where(sel, gy0, 0.0), axis=1, keepdims=True)
    tbx1 = jnp.sum(jnp.where(sel, gx1, 0.0), axis=1, keepdims=True)
    tby1 = jnp.sum(jnp.where(sel, gy1, 0.0), axis=1, keepdims=True)
    q = maxiou * posf

    p = cls_ref[...]
    cio = jax.lax.broadcasted_iota(jnp.int32, (BNM, N_CLS), 1)
    t = jnp.where(cio == label, q, 0.0)
    bce = (jnp.maximum(p, 0.0) - p * t
           + jnp.log(1.0 + jnp.exp(-jnp.abs(p))))
    bce_sum = jnp.sum(bce)

    ltrb = (jnp.concatenate(
        [acx - tbx0, acy - tby0, tbx1 - acx, tby1 - acy], axis=1)
        / STRIDE)
    tgt = jnp.clip(ltrb, 0.0, N_BINS - 1 - 1e-3)
    r64 = reg_ref[...]
    bio = jax.lax.broadcasted_iota(jnp.int32, (BNM, 4 * N_BINS), 1)
    reg_sum = jnp.float32(0.0)
    for g in range(4):
        seg = (bio >= g * N_BINS) & (bio < (g + 1) * N_BINS)
        tg = tgt[:, g : g + 1]
        left = jnp.clip(jnp.floor(tg).astype(jnp.int32), 0, N_BINS - 1)
        right = jnp.clip(left + 1, 0, N_BINS - 1)
        wr = tg - left.astype(jnp.float32)
        wl = 1.0 - wr
        mg = jnp.max(jnp.where(seg, r64, -BIGF), axis=1, keepdims=True)
        sg = jnp.sum(jnp.where(seg, jnp.exp(r64 - mg), 0.0),
                     axis=1, keepdims=True)
        lse = jnp.log(sg) + mg
        binid = bio - g * N_BINS
        p_l = jnp.sum(jnp.where(seg & (binid == left), r64, 0.0),
                      axis=1, keepdims=True)
        p_r = jnp.sum(jnp.where(seg & (binid == right), r64, 0.0),
                      axis=1, keepdims=True)
        elem = (lse - p_l) * wl + (lse - p_r) * wr
        reg_sum = reg_sum + jnp.sum(elem * posf)

    npos = jnp.sum(posf)
    lane = jax.lax.broadcasted_iota(jnp.int32, (8, MPAD), 1)
    row = jax.lax.broadcasted_iota(jnp.int32, (8, MPAD), 0)
    contrib = jnp.where((row == 0) & (lane == 0), bce_sum, 0.0)
    contrib = contrib + jnp.where((row == 0) & (lane == 1), reg_sum, 0.0)
    contrib = contrib + jnp.where((row == 0) & (lane == 2), npos, 0.0)

    @pl.when(i == 0)
    def _():
        out_ref[...] = jnp.zeros(out_ref.shape, jnp.float32)

    out_ref[...] += contrib


def _main_call(anchors, cls_preds, reg64, gt_t, lab_row, thresh):
    return pl.pallas_call(
        _main_kernel,
        grid=(NBLKM,),
        in_specs=[
            pl.BlockSpec((BNM, 4), lambda b: (b, 0)),
            pl.BlockSpec((BNM, N_CLS), lambda b: (b, 0)),
            pl.BlockSpec((BNM, 4 * N_BINS), lambda b: (b, 0)),
            pl.BlockSpec((8, MPAD), lambda b: (0, 0)),
            pl.BlockSpec((8, MPAD), lambda b: (0, 0)),
            pl.BlockSpec((8, MPAD), lambda b: (0, 0)),
        ],
        out_specs=pl.BlockSpec((8, MPAD), lambda b: (0, 0)),
        out_shape=jax.ShapeDtypeStruct((8, MPAD), jnp.float32),
        compiler_params=pltpu.CompilerParams(
            dimension_semantics=("arbitrary",)),
    )(anchors, cls_preds, reg64, gt_t, lab_row, thresh)


def kernel(cls_preds, reg_preds, anchors, gt_boxes, gt_labels):
    M = gt_boxes.shape[0]
    far = jnp.float32(2.0e9)
    pad = jnp.full((MPAD - M, 4), far, gt_boxes.dtype)
    gt_pad = jnp.concatenate([gt_boxes, pad], axis=0)
    gt_t = jnp.zeros((8, MPAD), jnp.float32).at[0:4, :].set(gt_pad.T)
    lab_pad = jnp.concatenate(
        [gt_labels.astype(jnp.int32), jnp.zeros((MPAD - M,), jnp.int32)])
    lab_row = jnp.zeros((8, MPAD), jnp.int32).at[0, :].set(lab_pad)

    thresh = _thresh_call(anchors, gt_t)
    reg64 = reg_preds.reshape(N_ANCH, 4 * N_BINS)
    acc = _main_call(anchors, cls_preds, reg64, gt_t, lab_row, thresh)
    bce_sum = acc[0, 0]
    reg_sum = acc[0, 1]
    npos = jnp.maximum(acc[0, 2], 1.0)
    return bce_sum / npos + reg_sum / (npos * 4.0)


# speedup vs baseline: 3.8503x; 1.5566x over previous
import jax
import jax.numpy as jnp
from jax.experimental import pallas as pl
from jax.experimental.pallas import tpu as pltpu

N_ANCH = 20000
N_CLS = 80
N_BINS = 16
TOP_K = 9
STRIDE = 8.0
MPAD = 128
BN0 = 400
NBLK0 = N_ANCH // BN0
BNM = 1000
NBLKM = N_ANCH // BNM
BIGF = 3.0e38
BIGI = 2**30


def _dist_iou(ax0, ay0, ax1, ay1, gx0, gy0, gx1, gy1):
    acx = (ax0 + ax1) * 0.5
    acy = (ay0 + ay1) * 0.5
    gcx = (gx0 + gx1) * 0.5
    gcy = (gy0 + gy1) * 0.5
    dx = acx - gcx
    dy = acy - gcy
    dist = jnp.sqrt(dx * dx + dy * dy)
    area_a = (ax1 - ax0) * (ay1 - ay0)
    area_g = (gx1 - gx0) * (gy1 - gy0)
    ltx = jnp.maximum(ax0, gx0)
    lty = jnp.maximum(ay0, gy0)
    rbx = jnp.minimum(ax1, gx1)
    rby = jnp.minimum(ay1, gy1)
    iw = jnp.maximum(rbx - ltx, 0.0)
    ih = jnp.maximum(rby - lty, 0.0)
    inter = iw * ih
    union = area_a + area_g - inter
    iou = inter / jnp.maximum(union, 1e-9)
    return dist, iou, acx, acy


def _thresh_kernel(a_ref, gt_ref, out_ref, dcand, icand, ucand):
    b = pl.program_id(0)

    @pl.when(b == 0)
    def _init():
        dcand[...] = jnp.full(dcand.shape, BIGF, jnp.float32)
        icand[...] = jnp.full(icand.shape, BIGI, jnp.int32)
        ucand[...] = jnp.zeros(ucand.shape, jnp.float32)

    ax0 = a_ref[:, 0:1]
    ay0 = a_ref[:, 1:2]
    ax1 = a_ref[:, 2:3]
    ay1 = a_ref[:, 3:4]
    gx0 = gt_ref[0:1, :]
    gy0 = gt_ref[1:2, :]
    gx1 = gt_ref[2:3, :]
    gy1 = gt_ref[3:4, :]
    dist, iou, _, _ = _dist_iou(ax0, ay0, ax1, ay1, gx0, gy0, gx1, gy1)
    rows = jax.lax.broadcasted_iota(jnp.int32, (BN0, MPAD), 0) + b * BN0
    d = dist
    for r in range(TOP_K):
        mv = jnp.min(d, axis=0, keepdims=True)
        ism = d == mv
        midx = jnp.min(jnp.where(ism, rows, BIGI), axis=0, keepdims=True)
        sel = rows == midx
        uv = jnp.sum(jnp.where(sel, iou, 0.0), axis=0, keepdims=True)
        dcand[b, r : r + 1, :] = mv
        icand[b, r : r + 1, :] = midx
        ucand[b, r : r + 1, :] = uv
        d = jnp.where(sel, BIGF, d)

    @pl.when(b == NBLK0 - 1)
    def _merge():
        dd = dcand[...].reshape(NBLK0 * 16, MPAD)
        ii = icand[...].reshape(NBLK0 * 16, MPAD)
        uu = ucand[...].reshape(NBLK0 * 16, MPAD)
        picks = []
        d2 = dd
        for _ in range(TOP_K):
            mv = jnp.min(d2, axis=0, keepdims=True)
            tie = d2 == mv
            midx = jnp.min(jnp.where(tie, ii, BIGI), axis=0, keepdims=True)
            sel = tie & (ii == midx)
            uv = jnp.sum(jnp.where(sel, uu, 0.0), axis=0, keepdims=True)
            picks.append(uv)
            d2 = jnp.where(sel, BIGF, d2)
        s = picks[0]
        for x in picks[1:]:
            s = s + x
        mean = s / jnp.float32(TOP_K)
        var = (picks[0] - mean) ** 2
        for x in picks[1:]:
            var = var + (x - mean) ** 2
        std = jnp.sqrt(var / jnp.float32(TOP_K - 1))
        out_ref[...] = jnp.broadcast_to(mean + std, out_ref.shape)


def _thresh_call(anchors, gt_t):
    return pl.pallas_call(
        _thresh_kernel,
        grid=(NBLK0,),
        in_specs=[
            pl.BlockSpec((BN0, 4), lambda b: (b, 0)),
            pl.BlockSpec((8, MPAD), lambda b: (0, 0)),
        ],
        out_specs=pl.BlockSpec((8, MPAD), lambda b: (0, 0)),
        out_shape=jax.ShapeDtypeStruct((8, MPAD), jnp.float32),
        scratch_shapes=[
            pltpu.VMEM((NBLK0, 16, MPAD), jnp.float32),
            pltpu.VMEM((NBLK0, 16, MPAD), jnp.int32),
            pltpu.VMEM((NBLK0, 16, MPAD), jnp.float32),
        ],
        compiler_params=pltpu.CompilerParams(
            dimension_semantics=("arbitrary",)),
    )(anchors, gt_t)


def _main_kernel(a_ref, cls_ref, reg_ref, gt_ref, tbl_ref, gmask_ref,
                 th_ref, out_ref):
    i = pl.program_id(0)
    ax0 = a_ref[:, 0:1]
    ay0 = a_ref[:, 1:2]
    ax1 = a_ref[:, 2:3]
    ay1 = a_ref[:, 3:4]
    gx0 = gt_ref[0:1, :]
    gy0 = gt_ref[1:2, :]
    gx1 = gt_ref[2:3, :]
    gy1 = gt_ref[3:4, :]
    _, iou, acx, acy = _dist_iou(ax0, ay0, ax1, ay1, gx0, gy0, gx1, gy1)
    thresh = th_ref[0:1, :]
    inside = ((acx >= gx0) & (acx <= gx1) & (acy >= gy0) & (acy <= gy1))
    pos = (iou >= thresh) & inside
    lanem = jax.lax.broadcasted_iota(jnp.int32, (BNM, MPAD), 1)
    matched = jnp.max(jnp.where(pos, lanem, -1), axis=1, keepdims=True)
    sel = lanem == matched
    maxiou = jnp.sum(jnp.where(sel, iou, 0.0), axis=1, keepdims=True)
    gath = jnp.dot(sel.astype(jnp.float32), tbl_ref[...],
                   preferred_element_type=jnp.float32)
    tbx0 = gath[:, 0:1]
    tby0 = gath[:, 1:2]
    tbx1 = gath[:, 2:3]
    tby1 = gath[:, 3:4]
    label = gath[:, 4:5]
    posf = gath[:, 5:6]
    q = maxiou * posf

    p = cls_ref[...]
    cio = jax.lax.broadcasted_iota(
        jnp.int32, (BNM, N_CLS), 1).astype(jnp.float32)
    t = jnp.where(cio == label, q, 0.0)
    bce = (jnp.maximum(p, 0.0) - p * t
           + jnp.log(1.0 + jnp.exp(-jnp.abs(p))))
    bce_row = jnp.sum(bce, axis=0, keepdims=True)

    tl = (acx - tbx0) / STRIDE
    tt = (acy - tby0) / STRIDE
    tr = (tbx1 - acx) / STRIDE
    tb = (tby1 - acy) / STRIDE
    r64 = reg_ref[...]
    bio = jax.lax.broadcasted_iota(jnp.int32, (BNM, 4 * N_BINS), 1)
    gid = bio // N_BINS
    binf = (bio % N_BINS).astype(jnp.float32)
    tgt64 = jnp.where(gid == 0, tl,
                      jnp.where(gid == 1, tt,
                                jnp.where(gid == 2, tr, tb)))
    tgt64 = jnp.clip(tgt64, 0.0, N_BINS - 1 - 1e-3)
    w = jnp.maximum(1.0 - jnp.abs(binf - tgt64), 0.0)
    ex = jnp.exp(r64)
    gm = gmask_ref[...]
    s4 = jnp.dot(ex, gm, preferred_element_type=jnp.float32)[:, 0:4]
    t4 = jnp.dot(r64 * w, gm, preferred_element_type=jnp.float32)[:, 0:4]
    elem4 = (jnp.log(s4) - t4) * posf
    reg_row = jnp.sum(elem4, axis=0, keepdims=True)
    npos_row = jnp.sum(posf, axis=0, keepdims=True)

    @pl.when(i == 0)
    def _():
        out_ref[...] = jnp.zeros(out_ref.shape, jnp.float32)

    out_ref[0:1, 0:N_CLS] += bce_row
    out_ref[1:2, 0:4] += reg_row
    out_ref[2:3, 0:1] += npos_row


def _main_call(anchors, cls_preds, reg64, gt_t, tbl, gmask, thresh):
    return pl.pallas_call(
        _main_kernel,
        grid=(NBLKM,),
        in_specs=[
            pl.BlockSpec((BNM, 4), lambda b: (b, 0)),
            pl.BlockSpec((BNM, N_CLS), lambda b: (b, 0)),
            pl.BlockSpec((BNM, 4 * N_BINS), lambda b: (b, 0)),
            pl.BlockSpec((8, MPAD), lambda b: (0, 0)),
            pl.BlockSpec((MPAD, MPAD), lambda b: (0, 0)),
            pl.BlockSpec((4 * N_BINS, MPAD), lambda b: (0, 0)),
            pl.BlockSpec((8, MPAD), lambda b: (0, 0)),
        ],
        out_specs=pl.BlockSpec((8, MPAD), lambda b: (0, 0)),
        out_shape=jax.ShapeDtypeStruct((8, MPAD), jnp.float32),
        compiler_params=pltpu.CompilerParams(
            dimension_semantics=("arbitrary",)),
    )(anchors, cls_preds, reg64, gt_t, tbl, gmask, thresh)


def kernel(cls_preds, reg_preds, anchors, gt_boxes, gt_labels):
    M = gt_boxes.shape[0]
    far = jnp.float32(2.0e9)
    pad = jnp.full((MPAD - M, 4), far, gt_boxes.dtype)
    gt_pad = jnp.concatenate([gt_boxes, pad], axis=0)
    gt_t = jnp.zeros((8, MPAD), jnp.float32).at[0:4, :].set(gt_pad.T)
    lab_pad = jnp.concatenate(
        [gt_labels.astype(jnp.float32), jnp.zeros((MPAD - M,), jnp.float32)])
    tbl = jnp.zeros((MPAD, MPAD), jnp.float32)
    tbl = tbl.at[:, 0:4].set(gt_pad)
    tbl = tbl.at[:, 4].set(lab_pad)
    tbl = tbl.at[:, 5].set(1.0)
    bidx = jnp.arange(4 * N_BINS)
    gmask = (jnp.arange(MPAD)[None, :] == (bidx // N_BINS)[:, None]
             ).astype(jnp.float32)

    thresh = _thresh_call(anchors, gt_t)
    reg64 = reg_preds.reshape(N_ANCH, 4 * N_BINS)
    acc = _main_call(anchors, cls_preds, reg64, gt_t, tbl, gmask, thresh)
    bce_sum = jnp.sum(acc[0, :])
    reg_sum = jnp.sum(acc[1, :])
    npos = jnp.maximum(acc[2, 0], 1.0)
    return bce_sum / npos + reg_sum / (npos * 4.0)
